# baseline (speedup 1.0000x reference)
import functools
import math

import jax
import jax.numpy as jnp
from jax import lax
from jax.experimental import pallas as pl
from jax.experimental.pallas import tpu as pltpu
from jax.experimental.pallas import tpu_sc as plsc

_VOCAB = 1000000
_DIM = 64
_B = 4096
_L = 200

_NC = 2
_NS = 16
_NW = _NC * _NS

_IW = 100
_XROWS = _B * _L // _IW
_CP = 4
_ROWS_PER_W = _XROWS // _NW
_CHUNKS = _ROWS_PER_W // _CP


def _pos_table():
    def body(o_ref):
        i = lax.broadcasted_iota(jnp.int32, (_L, _DIM), 0).astype(jnp.float32)
        d = lax.broadcasted_iota(jnp.int32, (_L, _DIM), 1)
        d_even = ((d // 2) * 2).astype(jnp.float32)
        angle = i * jnp.exp(d_even * (-math.log(10000.0) / _DIM))
        o_ref[...] = jnp.where(d % 2 == 0, jnp.sin(angle), jnp.cos(angle))

    return pl.pallas_call(
        body, out_shape=jax.ShapeDtypeStruct((_L, _DIM), jnp.float32)
    )()


@functools.partial(
    pl.kernel,
    mesh=plsc.VectorSubcoreMesh(core_axis_name="c", subcore_axis_name="s"),
    out_type=jax.ShapeDtypeStruct((_XROWS, _IW, _DIM), jnp.float32),
    scratch_types=[
        pltpu.VMEM((_CP, _IW), jnp.int32),
        pltpu.VMEM((_CP, _IW, _DIM), jnp.float32),
        pltpu.VMEM((_CP, _IW, _DIM), jnp.float32),
        pltpu.SemaphoreType.DMA,
    ],
    compiler_params=pltpu.CompilerParams(use_tc_tiling_on_sc=False),
)
def _sc_lookup(table_hbm, x_hbm, pos_hbm, out_hbm, idx_v, rows_v, pos_v, sem):
    wid = lax.axis_index("s") * _NC + lax.axis_index("c")
    base = wid * _ROWS_PER_W

    pltpu.sync_copy(pos_hbm, pos_v)

    def chunk_body(t, carry):
        xrow = base + t * _CP
        pltpu.sync_copy(x_hbm.at[pl.ds(xrow, _CP)], idx_v)
        descs = [
            pltpu.async_copy(table_hbm.at[idx_v.at[j]], rows_v.at[j], sem)
            for j in range(_CP)
        ]
        for d in descs:
            d.wait()

        for j in range(_CP):
            def add_body(i, c):
                for col in range(0, _DIM, 16):
                    seg = pl.ds(col, 16)
                    plsc.addupdate(rows_v.at[j, i, seg], pos_v[j, i, seg])
                return c

            lax.fori_loop(0, _IW, add_body, 0)

        pltpu.sync_copy(rows_v, out_hbm.at[pl.ds(xrow, _CP)])
        return carry

    lax.fori_loop(0, _CHUNKS, chunk_body, 0)


def kernel(x, table):
    pos = _pos_table()
    reps = _CP * _IW // _L
    pos_tiled = jnp.tile(pos, (reps, 1)).reshape(_CP, _IW, _DIM)
    x2d = x.reshape(_XROWS, _IW)
    out = _sc_lookup(table, x2d, pos_tiled)
    return out.reshape(_B, _L, _DIM)


# speedup vs baseline: 1.1509x; 1.1509x over previous
import functools
import math

import jax
import jax.numpy as jnp
from jax import lax
from jax.experimental import pallas as pl
from jax.experimental.pallas import tpu as pltpu
from jax.experimental.pallas import tpu_sc as plsc

_VOCAB = 1000000
_DIM = 64
_B = 4096
_L = 200

_NC = 2
_NS = 16
_NW = _NC * _NS

_IW = 100
_XROWS = _B * _L // _IW
_CP = 2
_NBUF = 4
_ROWS_PER_W = _XROWS // _NW
_CHUNKS = _ROWS_PER_W // _CP


def _pos_table():
    def body(o_ref):
        i = lax.broadcasted_iota(jnp.int32, (_L, _DIM), 0).astype(jnp.float32)
        d = lax.broadcasted_iota(jnp.int32, (_L, _DIM), 1)
        d_even = ((d // 2) * 2).astype(jnp.float32)
        angle = i * jnp.exp(d_even * (-math.log(10000.0) / _DIM))
        o_ref[...] = jnp.where(d % 2 == 0, jnp.sin(angle), jnp.cos(angle))

    return pl.pallas_call(
        body, out_shape=jax.ShapeDtypeStruct((_L, _DIM), jnp.float32)
    )()


@functools.partial(
    pl.kernel,
    mesh=plsc.VectorSubcoreMesh(core_axis_name="c", subcore_axis_name="s"),
    out_type=jax.ShapeDtypeStruct((_XROWS, _IW, _DIM), jnp.float32),
    scratch_types=[
        pltpu.VMEM((_ROWS_PER_W, _IW), jnp.int32),
        pltpu.VMEM((_NBUF, _CP, _IW, _DIM), jnp.float32),
        pltpu.VMEM((_CP, _IW, _DIM), jnp.float32),
        pltpu.SemaphoreType.DMA,
        pltpu.SemaphoreType.DMA,
    ],
    compiler_params=pltpu.CompilerParams(use_tc_tiling_on_sc=False),
)
def _sc_lookup(table_hbm, x_hbm, pos_hbm, out_hbm, idx_v, rows_v, pos_v,
               gsem, osem):
    wid = lax.axis_index("s") * _NC + lax.axis_index("c")
    base = wid * _ROWS_PER_W

    pltpu.sync_copy(pos_hbm, pos_v)
    pltpu.sync_copy(x_hbm.at[pl.ds(base, _ROWS_PER_W)], idx_v)

    def fire_gather(t, b):
        for j in range(_CP):
            pltpu.async_copy(
                table_hbm.at[idx_v.at[t * _CP + j]], rows_v.at[b, j], gsem
            )

    def wait_gather(b):
        for j in range(_CP):
            pltpu.make_async_copy(
                table_hbm.at[idx_v.at[0]], rows_v.at[b, j], gsem
            ).wait()

    def fire_out(t, b):
        pltpu.async_copy(
            rows_v.at[b], out_hbm.at[pl.ds(base + t * _CP, _CP)], osem
        )

    def wait_out():
        pltpu.make_async_copy(
            rows_v.at[0], out_hbm.at[pl.ds(base, _CP)], osem
        ).wait()

    def add_pos(b):
        for j in range(_CP):
            @plsc.parallel_loop(0, _IW, 1, unroll=4)
            def _(i):
                for col in range(0, _DIM, 16):
                    seg = pl.ds(col, 16)
                    plsc.addupdate(rows_v.at[b, j, i, seg], pos_v[j, i, seg])

    def step(t, b, fire_ahead):
        if fire_ahead:
            wait_out()
            fire_gather(t + 2, (b + 2) % _NBUF)
        wait_gather(b)
        add_pos(b)
        fire_out(t, b)

    pltpu.async_copy(rows_v.at[2], out_hbm.at[pl.ds(base, _CP)], osem)
    pltpu.async_copy(rows_v.at[3], out_hbm.at[pl.ds(base + _CP, _CP)], osem)
    fire_gather(0, 0)
    fire_gather(1, 1)

    for t in range(4):
        step(t, t % _NBUF, True)

    def group(g, carry):
        t0 = 4 + g * _NBUF
        for k in range(_NBUF):
            step(t0 + k, k, True)
        return carry

    lax.fori_loop(0, (_CHUNKS - 8) // _NBUF, group, 0)

    for t in range(_CHUNKS - 4, _CHUNKS):
        step(t, t % _NBUF, t + 2 < _CHUNKS)

    for _ in range(4):
        wait_out()


def kernel(x, table):
    pos = _pos_table()
    pos_blk = pos.reshape(_CP, _IW, _DIM)
    x2d = x.reshape(_XROWS, _IW)
    out = _sc_lookup(table, x2d, pos_blk)
    return out.reshape(_B, _L, _DIM)
